# baseline (speedup 1.0000x reference)
import functools
import jax
import jax.numpy as jnp
from jax import lax
from jax.experimental import pallas as pl
from jax.experimental.pallas import tpu as pltpu
from jax.experimental.pallas import tpu_sc as plsc

_NC = 2
_NS = 16
_NW = _NC * _NS
_ROWS_OUT = 64
_ROWS_PER_W = _ROWS_OUT // _NW
_D = 32768


@functools.partial(
    pl.kernel,
    mesh=plsc.VectorSubcoreMesh(core_axis_name="c", subcore_axis_name="s"),
    out_type=jax.ShapeDtypeStruct((_ROWS_OUT, _D), jnp.float32),
    scratch_types=(
        [pltpu.VMEM((_ROWS_PER_W, _D), jnp.float32)]
        + [pltpu.SemaphoreType.DMA] * 11
    ),
)
def _sc_gather(u_hbm, out_hbm, buf, *sems):
    wid = lax.axis_index("s") * _NC + lax.axis_index("c")
    base = wid * _ROWS_PER_W
    sizes = (2048, 2048, 4096, 8192, 16384)
    chunks = []
    for r in range(_ROWS_PER_W):
        off = 0
        for sz in sizes:
            chunks.append((base + r, off, sz))
            off += sz
    ins = []
    for k, (row, off, sz) in enumerate(chunks):
        cp = pltpu.make_async_copy(
            u_hbm.at[4 * row, pl.ds(off, sz)],
            buf.at[row - base, pl.ds(off, sz)], sems[k])
        cp.start()
        ins.append(cp)
    outs = []
    for k, (row, off, sz) in enumerate(chunks):
        ins[k].wait()
        cp = pltpu.make_async_copy(
            buf.at[row - base, pl.ds(off, sz)],
            out_hbm.at[row, pl.ds(off, sz)], sems[10])
        cp.start()
        outs.append(cp)
    for cp in outs:
        cp.wait()


def kernel(u):
    return _sc_gather(u)


# speedup vs baseline: 1.0225x; 1.0225x over previous
import functools
import jax
import jax.numpy as jnp
from jax import lax
from jax.experimental import pallas as pl
from jax.experimental.pallas import tpu as pltpu
from jax.experimental.pallas import tpu_sc as plsc

_NC = 2
_NS = 16
_NW = _NC * _NS
_ROWS_OUT = 64
_ROWS_PER_W = _ROWS_OUT // _NW
_D = 32768


@functools.partial(
    pl.kernel,
    mesh=plsc.VectorSubcoreMesh(core_axis_name="c", subcore_axis_name="s"),
    out_type=jax.ShapeDtypeStruct((_ROWS_OUT, _D), jnp.float32),
    scratch_types=[
        pltpu.VMEM((8, _D // 4), jnp.float32),
        pltpu.SemaphoreType.DMA,
        pltpu.SemaphoreType.DMA,
        pltpu.SemaphoreType.DMA,
        pltpu.SemaphoreType.DMA,
        pltpu.SemaphoreType.DMA,
        pltpu.SemaphoreType.DMA,
        pltpu.SemaphoreType.DMA,
        pltpu.SemaphoreType.DMA,
        pltpu.SemaphoreType.DMA,
    ],
)
def _sc_gather(u_hbm, out_hbm, buf, *sems):
    wid = lax.axis_index("s") * _NC + lax.axis_index("c")
    base = wid * _ROWS_PER_W
    q = _D // 4
    chunks = [(base + r, h) for r in range(_ROWS_PER_W) for h in range(4)]
    ins = []
    for k, (row, h) in enumerate(chunks):
        cp = pltpu.make_async_copy(
            u_hbm.at[4 * row, pl.ds(h * q, q)], buf.at[k], sems[k])
        cp.start()
        ins.append(cp)
    outs = []
    for k, (row, h) in enumerate(chunks):
        ins[k].wait()
        cp = pltpu.make_async_copy(
            buf.at[k], out_hbm.at[row, pl.ds(h * q, q)], sems[8])
        cp.start()
        outs.append(cp)
    for cp in outs:
        cp.wait()


def kernel(u):
    return _sc_gather(u)
